# Initial kernel scaffold; baseline (speedup 1.0000x reference)
import jax
import jax.numpy as jnp
from jax.experimental import pallas as pl


def kernel(x, split, W, b):
    raise NotImplementedError("write your pallas kernel here")


# speedup vs baseline: 9.2412x; 9.2412x over previous
import functools

import jax
import jax.numpy as jnp
from jax.experimental import pallas as pl
from jax.experimental.pallas import tpu as pltpu

_TILE_M = 256


def _gmm_body(g_t, g_e, g_start, g_end, x_ref, w_ref, b_ref, o_ref, *, tile_m):
    g = pl.program_id(0)
    y = jnp.dot(x_ref[...], w_ref[0], preferred_element_type=jnp.float32)
    y = y + b_ref[0]
    row = g_t[g] * tile_m + jax.lax.broadcasted_iota(jnp.int32, (tile_m, 1), 0)
    mask = (row >= g_start[g]) & (row < g_end[g])
    o_ref[...] = jnp.where(mask, y, o_ref[...])


def _grouped_matmul(x_sorted, W, b3, g_t, g_e, g_start, g_end):
    n, d = x_sorted.shape
    e = W.shape[0]
    tile_m = _TILE_M
    grid = (g_t.shape[0],)
    grid_spec = pltpu.PrefetchScalarGridSpec(
        num_scalar_prefetch=4,
        grid=grid,
        in_specs=[
            pl.BlockSpec((tile_m, d), lambda g, gt, ge, gs, gn: (gt[g], 0)),
            pl.BlockSpec((1, d, d), lambda g, gt, ge, gs, gn: (ge[g], 0, 0)),
            pl.BlockSpec((1, 1, d), lambda g, gt, ge, gs, gn: (ge[g], 0, 0)),
        ],
        out_specs=pl.BlockSpec((tile_m, d), lambda g, gt, ge, gs, gn: (gt[g], 0)),
    )
    return pl.pallas_call(
        functools.partial(_gmm_body, tile_m=tile_m),
        grid_spec=grid_spec,
        out_shape=jax.ShapeDtypeStruct((n, d), jnp.float32),
    )(g_t, g_e, g_start, g_end, x_sorted, W, b3)


def _work_units(offs, n, e, tile_m):
    t = n // tile_m
    g_max = t + e
    tt = jnp.arange(t, dtype=jnp.int32)[:, None]
    lo = offs[:-1][None, :]
    hi = offs[1:][None, :]
    inter = (lo < (tt + 1) * tile_m) & (hi > tt * tile_m)
    idx = jnp.nonzero(inter.reshape(-1), size=g_max, fill_value=t * e)[0]
    idx = idx.astype(jnp.int32)
    valid = idx < t * e
    g_t = jnp.minimum(idx // e, t - 1)
    g_e = jnp.minimum(idx % e, e - 1)
    g_start = jnp.where(valid, jnp.maximum(g_t * tile_m, offs[g_e]), 0)
    g_end = jnp.where(valid, jnp.minimum((g_t + 1) * tile_m, offs[g_e + 1]), 0)
    return g_t, g_e, g_start, g_end


def kernel(x, split, W, b):
    n, d = x.shape
    e = W.shape[0]
    split = split.astype(jnp.int32)

    perm = jnp.argsort(split, stable=True)
    counts = jnp.bincount(split, length=e)
    offs = jnp.concatenate(
        [jnp.zeros((1,), jnp.int32), jnp.cumsum(counts).astype(jnp.int32)]
    )
    x_sorted = x[perm]

    g_t, g_e, g_start, g_end = _work_units(offs, n, e, _TILE_M)
    y_sorted = _grouped_matmul(x_sorted, W, b.reshape(e, 1, d), g_t, g_e, g_start, g_end)

    inv = jnp.argsort(perm)
    return y_sorted[inv]
